# baseline (speedup 1.0000x reference)
import jax
import jax.numpy as jnp
from jax.experimental import pallas as pl
from jax.experimental.pallas import tpu as pltpu

_TM = 512


def _router_body(x_ref, w_ref, b_ref, o_ref):
    xb = x_ref[...].astype(jnp.bfloat16)
    wb = w_ref[...].astype(jnp.bfloat16)
    acc = jax.lax.dot_general(
        xb,
        wb,
        dimension_numbers=(((1,), (1,)), ((), ())),
        preferred_element_type=jnp.float32,
    )
    o_ref[...] = acc + b_ref[...]


def kernel(hidden_states, W, b):
    tokens, hidden = hidden_states.shape
    experts = W.shape[0]
    b2 = b.reshape(1, experts)
    return pl.pallas_call(
        _router_body,
        grid=(tokens // _TM,),
        in_specs=[
            pl.BlockSpec((_TM, hidden), lambda i: (i, 0)),
            pl.BlockSpec((experts, hidden), lambda i: (0, 0)),
            pl.BlockSpec((1, experts), lambda i: (0, 0)),
        ],
        out_specs=pl.BlockSpec((_TM, experts), lambda i: (i, 0)),
        out_shape=jax.ShapeDtypeStruct((tokens, experts), jnp.float32),
        compiler_params=pltpu.CompilerParams(
            vmem_limit_bytes=100 * 1024 * 1024,
        ),
    )(hidden_states, W, b2)


# speedup vs baseline: 1.1122x; 1.1122x over previous
import jax
import jax.numpy as jnp
from jax.experimental import pallas as pl
from jax.experimental.pallas import tpu as pltpu

_TM = 1024


def _router_body(x_ref, w_ref, b_ref, o_ref, wbf_ref):
    @pl.when(pl.program_id(0) == 0)
    def _cache_w():
        wbf_ref[...] = w_ref[...].astype(jnp.bfloat16)

    xb = x_ref[...].astype(jnp.bfloat16)
    acc = jax.lax.dot_general(
        xb,
        wbf_ref[...],
        dimension_numbers=(((1,), (1,)), ((), ())),
        preferred_element_type=jnp.float32,
    )
    o_ref[...] = acc + b_ref[...]


def kernel(hidden_states, W, b):
    tokens, hidden = hidden_states.shape
    experts = W.shape[0]
    b2 = b.reshape(1, experts)
    return pl.pallas_call(
        _router_body,
        grid=(tokens // _TM,),
        in_specs=[
            pl.BlockSpec((_TM, hidden), lambda i: (i, 0)),
            pl.BlockSpec((experts, hidden), lambda i: (0, 0)),
            pl.BlockSpec((1, experts), lambda i: (0, 0)),
        ],
        out_specs=pl.BlockSpec((_TM, experts), lambda i: (i, 0)),
        out_shape=jax.ShapeDtypeStruct((tokens, experts), jnp.float32),
        scratch_shapes=[pltpu.VMEM((experts, hidden), jnp.bfloat16)],
    )(hidden_states, W, b2)


# speedup vs baseline: 1.1177x; 1.0050x over previous
import jax
import jax.numpy as jnp
from jax.experimental import pallas as pl
from jax.experimental.pallas import tpu as pltpu

_TM = 1024


def _router_body(xl_ref, xr_ref, w_ref, b_ref, o_ref):
    kh = xl_ref.shape[1]
    wb = w_ref[...].astype(jnp.bfloat16)
    dn = (((1,), (1,)), ((), ()))
    accl = jax.lax.dot_general(
        xl_ref[...].astype(jnp.bfloat16), wb[:, :kh],
        dimension_numbers=dn, preferred_element_type=jnp.float32)
    accr = jax.lax.dot_general(
        xr_ref[...].astype(jnp.bfloat16), wb[:, kh:],
        dimension_numbers=dn, preferred_element_type=jnp.float32)
    o_ref[...] = accl + accr + b_ref[...]


def kernel(hidden_states, W, b):
    tokens, hidden = hidden_states.shape
    experts = W.shape[0]
    kh = hidden // 2
    b2 = b.reshape(1, experts)
    return pl.pallas_call(
        _router_body,
        grid=(tokens // _TM,),
        in_specs=[
            pl.BlockSpec((_TM, kh), lambda i: (i, 0)),
            pl.BlockSpec((_TM, kh), lambda i: (i, 1)),
            pl.BlockSpec((experts, hidden), lambda i: (0, 0)),
            pl.BlockSpec((1, experts), lambda i: (0, 0)),
        ],
        out_specs=pl.BlockSpec((_TM, experts), lambda i: (i, 0)),
        out_shape=jax.ShapeDtypeStruct((tokens, experts), jnp.float32),
    )(hidden_states, hidden_states, W, b2)
